# Initial kernel scaffold; baseline (speedup 1.0000x reference)
import functools

import jax
import jax.numpy as jnp
from jax import lax
from jax.experimental import pallas as pl
from jax.experimental.pallas import tpu as pltpu
from jax.experimental.pallas import tpu_sc as plsc

IMG = 224
HW = IMG * IMG
N = 100000
B = 16
C = 10000
G = C // 16
NCHUNK = N // C


def _floor(v):
    t = v.astype(jnp.int32)
    tf = t.astype(jnp.float32)
    return jnp.where(tf > v, t - 1, t)


def _splat_body(pts_hbm, m_hbm, out_hbm, s0, s1, buf, rbuf):
    cid = lax.axis_index("c")
    sid = lax.axis_index("s")
    wid = sid * 2 + cid

    @pl.when(wid < B)
    def _():
        zeros = jnp.zeros((16,), jnp.float32)

        def zbody(i, _):
            s0[pl.ds(i * 16, 16)] = zeros
            s1[pl.ds(i * 16, 16)] = zeros
            return 0

        lax.fori_loop(0, HW // 16, zbody, 0)

        pltpu.sync_copy(m_hbm.at[wid], rbuf)
        m = [
            plsc.load_gather(rbuf, [jnp.full((16,), j, jnp.int32)])
            for j in range(9)
        ]

        def group_body(g, car):
            zmn, zmx = car
            x = buf[0, pl.ds(g * 16, 16)]
            y = buf[1, pl.ds(g * 16, 16)]
            z = buf[2, pl.ds(g * 16, 16)]
            px = m[0] * x + m[1] * y + m[2] * z + 111.5
            py = m[3] * x + m[4] * y + m[5] * z + 111.5
            zc = m[6] * x + m[7] * y + m[8] * z
            zmn = jnp.minimum(zmn, zc)
            zmx = jnp.maximum(zmx, zc)
            px1i = _floor(px)
            py1i = _floor(py)
            px1f = px1i.astype(jnp.float32)
            py1f = py1i.astype(jnp.float32)
            fx = px - px1f
            fy = py - py1f
            gx = 1.0 - fx
            gy = 1.0 - fy
            px2i = px1i + 1
            py2i = py1i + 1
            mask = (px1i >= 0) & (py1i >= 0) & (px2i < IMG) & (py2i < IMG)
            x1 = jnp.clip(px1i, 0, IMG - 1)
            x2 = jnp.clip(px2i, 0, IMG - 1)
            y1 = jnp.clip(py1i, 0, IMG - 1) * IMG
            y2 = jnp.clip(py2i, 0, IMG - 1) * IMG
            i11 = y1 + x1
            i12 = y2 + x1
            i21 = y1 + x2
            i22 = y2 + x2
            w11 = gx * gy
            w12 = gx * fy
            w21 = fx * gy
            w22 = fx * fy
            plsc.addupdate_scatter(s0, [i11], w11, mask=mask)
            plsc.addupdate_scatter(s1, [i11], w11 * zc, mask=mask)
            plsc.addupdate_scatter(s0, [i12], w12, mask=mask)
            plsc.addupdate_scatter(s1, [i12], w12 * zc, mask=mask)
            plsc.addupdate_scatter(s0, [i21], w21, mask=mask)
            plsc.addupdate_scatter(s1, [i21], w21 * zc, mask=mask)
            plsc.addupdate_scatter(s0, [i22], w22, mask=mask)
            plsc.addupdate_scatter(s1, [i22], w22 * zc, mask=mask)
            return zmn, zmx

        def chunk_body(k, carry):
            pltpu.sync_copy(pts_hbm.at[wid, :, pl.ds(k * C, C)], buf)
            return lax.fori_loop(0, G, group_body, carry)

        zminv, zmaxv = lax.fori_loop(
            0,
            NCHUNK,
            chunk_body,
            (jnp.full((16,), jnp.inf), jnp.full((16,), -jnp.inf)),
        )

        zmin = jnp.min(zminv)
        zmax = jnp.max(zmaxv)
        denom = zmax - zmin + 1e-6
        a_vec = jnp.full((16,), 0.7) / jnp.full((16,), denom)
        c_vec = jnp.full((16,), 0.3) - a_vec * jnp.full((16,), zmin)

        def fin_body(i, _):
            sl = pl.ds(i * 16, 16)
            s0[sl] = a_vec * s1[sl] + c_vec * s0[sl]
            return 0

        lax.fori_loop(0, HW // 16, fin_body, 0)
        pltpu.sync_copy(s0, out_hbm.at[wid])


@jax.jit
def _render(pts_t, m):
    mesh = plsc.VectorSubcoreMesh(core_axis_name="c", subcore_axis_name="s")
    run = functools.partial(
        pl.kernel,
        out_type=jax.ShapeDtypeStruct((B, HW), jnp.float32),
        mesh=mesh,
        scratch_types=[
            pltpu.VMEM((HW,), jnp.float32),
            pltpu.VMEM((HW,), jnp.float32),
            pltpu.VMEM((3, C), jnp.float32),
            pltpu.VMEM((16,), jnp.float32),
        ],
    )(_splat_body)
    return run(pts_t, m)


def kernel(points, azimuth, elevation):
    cos_az, sin_az = jnp.cos(azimuth), jnp.sin(azimuth)
    cos_el, sin_el = jnp.cos(elevation), jnp.sin(elevation)
    z = jnp.zeros_like(cos_az)
    o = jnp.ones_like(cos_az)
    r_az = jnp.stack([
        jnp.stack([cos_az, z, sin_az], axis=-1),
        jnp.stack([z, o, z], axis=-1),
        jnp.stack([-sin_az, z, cos_az], axis=-1),
    ], axis=1)
    r_el = jnp.stack([
        jnp.stack([o, z, z], axis=-1),
        jnp.stack([z, cos_el, -sin_el], axis=-1),
        jnp.stack([z, sin_el, cos_el], axis=-1),
    ], axis=1)
    r = jnp.matmul(r_el, r_az)
    scale = jnp.array([112.0, 112.0, 1.0], jnp.float32)[None, :, None]
    m = (r * scale).reshape(B, 9)
    m = jnp.pad(m, ((0, 0), (0, 7)))
    pts_t = jnp.transpose(points, (0, 2, 1))
    img = _render(pts_t, m).reshape(B, IMG, IMG)
    return jnp.broadcast_to(img[:, None, :, :], (B, 3, IMG, IMG))


# speedup vs baseline: 24.1163x; 24.1163x over previous
import functools

import jax
import jax.numpy as jnp
from jax import lax
from jax.experimental import pallas as pl
from jax.experimental.pallas import tpu as pltpu
from jax.experimental.pallas import tpu_sc as plsc

IMG = 224
HW = IMG * IMG
N = 100000
B = 16
C = 10000
G = C // 16
NCHUNK = N // C


def _floor(v):
    t = v.astype(jnp.int32)
    tf = t.astype(jnp.float32)
    return jnp.where(tf > v, t - 1, t)


def _splat_body(pts_hbm, m_hbm, out_hbm, s0, s1, bufx, bufy, bufz, rbuf):
    cid = lax.axis_index("c")
    sid = lax.axis_index("s")
    wid = sid * 2 + cid

    @pl.when(wid < B)
    def _():
        zeros = jnp.zeros((16,), jnp.float32)

        def zbody(i, _):
            s0[pl.ds(i * 16, 16)] = zeros
            s1[pl.ds(i * 16, 16)] = zeros
            return 0

        lax.fori_loop(0, HW // 16, zbody, 0)

        pltpu.sync_copy(m_hbm.at[pl.ds(wid * 144, 144)], rbuf)
        m = [rbuf[pl.ds(j * 16, 16)] for j in range(9)]

        def group_body(g, car):
            zmn, zmx = car
            x = bufx[pl.ds(g * 16, 16)]
            y = bufy[pl.ds(g * 16, 16)]
            z = bufz[pl.ds(g * 16, 16)]
            rx = m[0] * x + m[1] * y + m[2] * z
            ry = m[3] * x + m[4] * y + m[5] * z
            zc = m[6] * x + m[7] * y + m[8] * z
            px = (rx + 1.0) * 112.0 - 0.5
            py = (ry + 1.0) * 112.0 - 0.5
            zmn = jnp.minimum(zmn, zc)
            zmx = jnp.maximum(zmx, zc)
            px1i = _floor(px)
            py1i = _floor(py)
            px1f = px1i.astype(jnp.float32)
            py1f = py1i.astype(jnp.float32)
            fx = px - px1f
            fy = py - py1f
            gx = 1.0 - fx
            gy = 1.0 - fy
            px2i = px1i + 1
            py2i = py1i + 1
            mask = (px1i >= 0) & (py1i >= 0) & (px2i < IMG) & (py2i < IMG)
            x1 = jnp.clip(px1i, 0, IMG - 1)
            x2 = jnp.clip(px2i, 0, IMG - 1)
            y1 = jnp.clip(py1i, 0, IMG - 1) * IMG
            y2 = jnp.clip(py2i, 0, IMG - 1) * IMG
            i11 = y1 + x1
            i12 = y2 + x1
            i21 = y1 + x2
            i22 = y2 + x2
            w11 = gx * gy
            w12 = gx * fy
            w21 = fx * gy
            w22 = fx * fy
            plsc.addupdate_scatter(s0, [i11], w11, mask=mask)
            plsc.addupdate_scatter(s1, [i11], w11 * zc, mask=mask)
            plsc.addupdate_scatter(s0, [i12], w12, mask=mask)
            plsc.addupdate_scatter(s1, [i12], w12 * zc, mask=mask)
            plsc.addupdate_scatter(s0, [i21], w21, mask=mask)
            plsc.addupdate_scatter(s1, [i21], w21 * zc, mask=mask)
            plsc.addupdate_scatter(s0, [i22], w22, mask=mask)
            plsc.addupdate_scatter(s1, [i22], w22 * zc, mask=mask)
            return zmn, zmx

        def chunk_body(k, carry):
            base = wid * (3 * N) + k * C
            pltpu.sync_copy(pts_hbm.at[pl.ds(base, C)], bufx)
            pltpu.sync_copy(pts_hbm.at[pl.ds(base + N, C)], bufy)
            pltpu.sync_copy(pts_hbm.at[pl.ds(base + 2 * N, C)], bufz)
            return lax.fori_loop(0, G, group_body, carry)

        zminv, zmaxv = lax.fori_loop(
            0,
            NCHUNK,
            chunk_body,
            (jnp.full((16,), jnp.inf), jnp.full((16,), -jnp.inf)),
        )

        zmin = jnp.min(zminv)
        zmax = jnp.max(zmaxv)
        denom = zmax - zmin + 1e-6
        a_vec = jnp.full((16,), 0.7) / jnp.full((16,), denom)
        c_vec = jnp.full((16,), 0.3) - a_vec * jnp.full((16,), zmin)

        def fin_body(i, _):
            sl = pl.ds(i * 16, 16)
            s0[sl] = a_vec * s1[sl] + c_vec * s0[sl]
            return 0

        lax.fori_loop(0, HW // 16, fin_body, 0)
        pltpu.sync_copy(s0, out_hbm.at[pl.ds(wid * HW, HW)])


@jax.jit
def _render(pts_t, m):
    mesh = plsc.VectorSubcoreMesh(core_axis_name="c", subcore_axis_name="s")
    run = functools.partial(
        pl.kernel,
        out_type=jax.ShapeDtypeStruct((B * HW,), jnp.float32),
        mesh=mesh,
        scratch_types=[
            pltpu.VMEM((HW,), jnp.float32),
            pltpu.VMEM((HW,), jnp.float32),
            pltpu.VMEM((C,), jnp.float32),
            pltpu.VMEM((C,), jnp.float32),
            pltpu.VMEM((C,), jnp.float32),
            pltpu.VMEM((144,), jnp.float32),
        ],
        compiler_params=pltpu.CompilerParams(needs_layout_passes=False),
    )(_splat_body)
    return run(pts_t, m)


def kernel(points, azimuth, elevation):
    cos_az, sin_az = jnp.cos(azimuth), jnp.sin(azimuth)
    cos_el, sin_el = jnp.cos(elevation), jnp.sin(elevation)
    z = jnp.zeros_like(cos_az)
    o = jnp.ones_like(cos_az)
    r_az = jnp.stack([
        jnp.stack([cos_az, z, sin_az], axis=-1),
        jnp.stack([z, o, z], axis=-1),
        jnp.stack([-sin_az, z, cos_az], axis=-1),
    ], axis=1)
    r_el = jnp.stack([
        jnp.stack([o, z, z], axis=-1),
        jnp.stack([z, cos_el, -sin_el], axis=-1),
        jnp.stack([z, sin_el, cos_el], axis=-1),
    ], axis=1)
    r = jnp.matmul(r_el, r_az)

    def bf16_round(v):
        u = lax.bitcast_convert_type(v, jnp.uint32)
        rr = (u + 0x7FFF + ((u >> 16) & 1)) & jnp.uint32(0xFFFF0000)
        return lax.bitcast_convert_type(rr, jnp.float32)

    m = bf16_round(r).reshape(B, 9)
    m = jnp.broadcast_to(m[:, :, None], (B, 9, 16)).reshape(-1)
    pts_t = bf16_round(jnp.transpose(points, (0, 2, 1))).reshape(-1)
    img = _render(pts_t, m).reshape(B, IMG, IMG)
    return jnp.broadcast_to(img[:, None, :, :], (B, 3, IMG, IMG))


# speedup vs baseline: 29.2260x; 1.2119x over previous
import functools

import jax
import jax.numpy as jnp
from jax import lax
from jax.experimental import pallas as pl
from jax.experimental.pallas import tpu as pltpu
from jax.experimental.pallas import tpu_sc as plsc

IMG = 224
HW = IMG * IMG
N = 100000
B = 16
C = 10000
G = C // 16
NPT = N // 2
HALF = HW // 2


def _floor(v):
    t = v.astype(jnp.int32)
    tf = t.astype(jnp.float32)
    return jnp.where(tf > v, t - 1, t)


def _splat_body(pts_hbm, m_hbm, out_hbm, s, bufx, bufy, bufz, rbuf,
                zbuf, shp, shz):
    cid = lax.axis_index("c")
    sid = lax.axis_index("s")
    view = cid * 8 + sid // 2
    half = sid % 2
    psid = sid ^ 1
    pt_base = view * (3 * N) + half * NPT

    zeros = jnp.zeros((16,), jnp.float32)

    def zbody(i, _):
        s[pl.ds(i * 16, 16)] = zeros
        return 0

    lax.fori_loop(0, HW // 16, zbody, 0)

    pltpu.sync_copy(m_hbm.at[pl.ds(view * 144, 144)], rbuf)
    m = [rbuf[pl.ds(j * 16, 16)] for j in range(9)]

    def zrange_group(g, car):
        zmn, zmx = car
        x = bufx[pl.ds(g * 16, 16)]
        y = bufy[pl.ds(g * 16, 16)]
        z = bufz[pl.ds(g * 16, 16)]
        zc = m[6] * x + m[7] * y + m[8] * z
        return jnp.minimum(zmn, zc), jnp.maximum(zmx, zc)

    def zrange_chunk(k, carry):
        base = pt_base + k * C
        pltpu.sync_copy(pts_hbm.at[pl.ds(base, C)], bufx)
        pltpu.sync_copy(pts_hbm.at[pl.ds(base + N, C)], bufy)
        pltpu.sync_copy(pts_hbm.at[pl.ds(base + 2 * N, C)], bufz)
        return lax.fori_loop(0, G, zrange_group, carry)

    zminv, zmaxv = lax.fori_loop(
        0,
        NPT // C,
        zrange_chunk,
        (jnp.full((16,), jnp.inf), jnp.full((16,), -jnp.inf)),
    )

    zbuf[pl.ds(0, 16)] = zminv
    zbuf[pl.ds(16, 16)] = zmaxv
    pltpu.sync_copy(zbuf, shz.at[pl.ds(sid * 32, 32)])
    plsc.subcore_barrier()
    pltpu.sync_copy(shz.at[pl.ds(psid * 32, 32)], zbuf)
    pzmin = zbuf[pl.ds(0, 16)]
    pzmax = zbuf[pl.ds(16, 16)]

    zmin = jnp.min(jnp.minimum(zminv, pzmin))
    zmax = jnp.max(jnp.maximum(zmaxv, pzmax))
    denom = zmax - zmin + 1e-6
    a_vec = jnp.full((16,), 0.7) / jnp.full((16,), denom)
    c_vec = jnp.full((16,), 0.3) - a_vec * jnp.full((16,), zmin)

    def group_body(g, _):
        x = bufx[pl.ds(g * 16, 16)]
        y = bufy[pl.ds(g * 16, 16)]
        z = bufz[pl.ds(g * 16, 16)]
        rx = m[0] * x + m[1] * y + m[2] * z
        ry = m[3] * x + m[4] * y + m[5] * z
        zc = m[6] * x + m[7] * y + m[8] * z
        px = (rx + 1.0) * 112.0 - 0.5
        py = (ry + 1.0) * 112.0 - 0.5
        feat = a_vec * zc + c_vec
        px1i = _floor(px)
        py1i = _floor(py)
        px1f = px1i.astype(jnp.float32)
        py1f = py1i.astype(jnp.float32)
        fx = px - px1f
        fy = py - py1f
        gx = 1.0 - fx
        gy = 1.0 - fy
        px2i = px1i + 1
        py2i = py1i + 1
        mask = (px1i >= 0) & (py1i >= 0) & (px2i < IMG) & (py2i < IMG)
        x1 = jnp.clip(px1i, 0, IMG - 1)
        x2 = jnp.clip(px2i, 0, IMG - 1)
        y1 = jnp.clip(py1i, 0, IMG - 1) * IMG
        y2 = jnp.clip(py2i, 0, IMG - 1) * IMG
        plsc.addupdate_scatter(s, [y1 + x1], feat * (gx * gy), mask=mask)
        plsc.addupdate_scatter(s, [y2 + x1], feat * (gx * fy), mask=mask)
        plsc.addupdate_scatter(s, [y1 + x2], feat * (fx * gy), mask=mask)
        plsc.addupdate_scatter(s, [y2 + x2], feat * (fx * fy), mask=mask)
        return 0

    def chunk_body(k, _):
        base = pt_base + k * C
        pltpu.sync_copy(pts_hbm.at[pl.ds(base, C)], bufx)
        pltpu.sync_copy(pts_hbm.at[pl.ds(base + N, C)], bufy)
        pltpu.sync_copy(pts_hbm.at[pl.ds(base + 2 * N, C)], bufz)
        return lax.fori_loop(0, G, group_body, 0)

    lax.fori_loop(0, NPT // C, chunk_body, 0)

    my_base = half * HALF
    ot_base = (1 - half) * HALF
    pltpu.sync_copy(s.at[pl.ds(ot_base, HALF)],
                    shp.at[pl.ds(sid * HALF, HALF)])
    plsc.subcore_barrier()

    for p0, ln in ((0, C), (C, C), (2 * C, HALF - 2 * C)):
        pltpu.sync_copy(shp.at[pl.ds(psid * HALF + p0, ln)],
                        bufx.at[pl.ds(0, ln)])

        def fin_body(i, _, p0=p0):
            sl = pl.ds(i * 16, 16)
            dst = pl.ds(my_base + p0 + i * 16, 16)
            s[dst] = s[dst] + bufx[sl]
            return 0

        lax.fori_loop(0, ln // 16, fin_body, 0)

    pltpu.sync_copy(s.at[pl.ds(my_base, HALF)],
                    out_hbm.at[pl.ds(view * HW + my_base, HALF)])


@jax.jit
def _render(pts_t, m):
    mesh = plsc.VectorSubcoreMesh(core_axis_name="c", subcore_axis_name="s")
    run = functools.partial(
        pl.kernel,
        out_type=jax.ShapeDtypeStruct((B * HW,), jnp.float32),
        mesh=mesh,
        scratch_types=[
            pltpu.VMEM((HW,), jnp.float32),
            pltpu.VMEM((C,), jnp.float32),
            pltpu.VMEM((C,), jnp.float32),
            pltpu.VMEM((C,), jnp.float32),
            pltpu.VMEM((144,), jnp.float32),
            pltpu.VMEM((32,), jnp.float32),
            pltpu.VMEM_SHARED((16 * HALF,), jnp.float32),
            pltpu.VMEM_SHARED((16 * 32,), jnp.float32),
        ],
        compiler_params=pltpu.CompilerParams(needs_layout_passes=False),
    )(_splat_body)
    return run(pts_t, m)


def kernel(points, azimuth, elevation):
    cos_az, sin_az = jnp.cos(azimuth), jnp.sin(azimuth)
    cos_el, sin_el = jnp.cos(elevation), jnp.sin(elevation)
    z = jnp.zeros_like(cos_az)
    o = jnp.ones_like(cos_az)
    r_az = jnp.stack([
        jnp.stack([cos_az, z, sin_az], axis=-1),
        jnp.stack([z, o, z], axis=-1),
        jnp.stack([-sin_az, z, cos_az], axis=-1),
    ], axis=1)
    r_el = jnp.stack([
        jnp.stack([o, z, z], axis=-1),
        jnp.stack([z, cos_el, -sin_el], axis=-1),
        jnp.stack([z, sin_el, cos_el], axis=-1),
    ], axis=1)
    r = jnp.matmul(r_el, r_az)

    def bf16_round(v):
        u = lax.bitcast_convert_type(v, jnp.uint32)
        rr = (u + 0x7FFF + ((u >> 16) & 1)) & jnp.uint32(0xFFFF0000)
        return lax.bitcast_convert_type(rr, jnp.float32)

    m = bf16_round(r).reshape(B, 9)
    m = jnp.broadcast_to(m[:, :, None], (B, 9, 16)).reshape(-1)
    pts_t = bf16_round(jnp.transpose(points, (0, 2, 1))).reshape(-1)
    img = _render(pts_t, m).reshape(B, IMG, IMG)
    return jnp.broadcast_to(img[:, None, :, :], (B, 3, IMG, IMG))


# speedup vs baseline: 46.0966x; 1.5772x over previous
import functools

import jax
import jax.numpy as jnp
from jax import lax
from jax.experimental import pallas as pl
from jax.experimental.pallas import tpu as pltpu
from jax.experimental.pallas import tpu_sc as plsc

IMG = 224
HW = IMG * IMG
N = 100000
B = 16
NP = 100096
C = 12512
G = C // 16
NPT = NP // 2
HALF = HW // 2


def _floor(v):
    t = v.astype(jnp.int32)
    tf = t.astype(jnp.float32)
    return jnp.where(tf > v, t - 1, t)


def _splat_body(pts_hbm, m_hbm, out_hbm, s, bufx, bufy, bufz, rbuf,
                zbuf, shp, shz):
    cid = lax.axis_index("c")
    sid = lax.axis_index("s")
    view = cid * 8 + sid // 2
    half = sid % 2
    psid = sid ^ 1
    pt_base = view * (3 * NP) + half * NPT
    pidx0 = half * NPT
    iota = lax.iota(jnp.int32, 16)
    inf16 = jnp.full((16,), jnp.inf)
    ninf16 = jnp.full((16,), -jnp.inf)

    zeros = jnp.zeros((16,), jnp.float32)

    def zbody(i, _):
        s[pl.ds(i * 16, 16)] = zeros
        return 0

    lax.fori_loop(0, HW // 16, zbody, 0)

    pltpu.sync_copy(m_hbm.at[pl.ds(view * 144, 144)], rbuf)
    m = [rbuf[pl.ds(j * 16, 16)] for j in range(9)]

    def zrange_chunk(k, carry):
        base = pt_base + k * C
        pltpu.sync_copy(pts_hbm.at[pl.ds(base, C)], bufx)
        pltpu.sync_copy(pts_hbm.at[pl.ds(base + NP, C)], bufy)
        pltpu.sync_copy(pts_hbm.at[pl.ds(base + 2 * NP, C)], bufz)
        kbase = pidx0 + k * C

        def zrange_group(g, car):
            zmn, zmx = car
            x = bufx[pl.ds(g * 16, 16)]
            y = bufy[pl.ds(g * 16, 16)]
            z = bufz[pl.ds(g * 16, 16)]
            zc = m[6] * x + m[7] * y + m[8] * z
            valid = (kbase + g * 16 + iota) < N
            zmn = jnp.minimum(zmn, jnp.where(valid, zc, inf16))
            zmx = jnp.maximum(zmx, jnp.where(valid, zc, ninf16))
            return zmn, zmx

        return lax.fori_loop(0, G, zrange_group, carry)

    zminv, zmaxv = lax.fori_loop(
        0,
        NPT // C,
        zrange_chunk,
        (jnp.full((16,), jnp.inf), jnp.full((16,), -jnp.inf)),
    )

    zbuf[pl.ds(0, 16)] = zminv
    zbuf[pl.ds(16, 16)] = zmaxv
    pltpu.sync_copy(zbuf, shz.at[pl.ds(sid * 32, 32)])
    plsc.subcore_barrier()
    pltpu.sync_copy(shz.at[pl.ds(psid * 32, 32)], zbuf)
    pzmin = zbuf[pl.ds(0, 16)]
    pzmax = zbuf[pl.ds(16, 16)]

    zmin = jnp.min(jnp.minimum(zminv, pzmin))
    zmax = jnp.max(jnp.maximum(zmaxv, pzmax))
    denom = zmax - zmin + 1e-6
    a_vec = jnp.full((16,), 0.7) / jnp.full((16,), denom)
    c_vec = jnp.full((16,), 0.3) - a_vec * jnp.full((16,), zmin)

    def group_body(g, kbase):
        x = bufx[pl.ds(g * 16, 16)]
        y = bufy[pl.ds(g * 16, 16)]
        z = bufz[pl.ds(g * 16, 16)]
        rx = m[0] * x + m[1] * y + m[2] * z
        ry = m[3] * x + m[4] * y + m[5] * z
        zc = m[6] * x + m[7] * y + m[8] * z
        px = (rx + 1.0) * 112.0 - 0.5
        py = (ry + 1.0) * 112.0 - 0.5
        feat = a_vec * zc + c_vec
        px1i = _floor(px)
        py1i = _floor(py)
        px1f = px1i.astype(jnp.float32)
        py1f = py1i.astype(jnp.float32)
        fx = px - px1f
        fy = py - py1f
        gx = 1.0 - fx
        gy = 1.0 - fy
        px2i = px1i + 1
        py2i = py1i + 1
        valid = (kbase + g * 16 + iota) < N
        mask = (px1i >= 0) & (py1i >= 0) & (px2i < IMG) & (py2i < IMG) & valid
        x1 = jnp.clip(px1i, 0, IMG - 1)
        x2 = jnp.clip(px2i, 0, IMG - 1)
        y1 = jnp.clip(py1i, 0, IMG - 1) * IMG
        y2 = jnp.clip(py2i, 0, IMG - 1) * IMG
        plsc.addupdate_scatter(s, [y1 + x1], feat * (gx * gy), mask=mask)
        plsc.addupdate_scatter(s, [y2 + x1], feat * (gx * fy), mask=mask)
        plsc.addupdate_scatter(s, [y1 + x2], feat * (fx * gy), mask=mask)
        plsc.addupdate_scatter(s, [y2 + x2], feat * (fx * fy), mask=mask)
        return kbase

    def chunk_body(k, _):
        base = pt_base + k * C
        pltpu.sync_copy(pts_hbm.at[pl.ds(base, C)], bufx)
        pltpu.sync_copy(pts_hbm.at[pl.ds(base + NP, C)], bufy)
        pltpu.sync_copy(pts_hbm.at[pl.ds(base + 2 * NP, C)], bufz)
        return lax.fori_loop(0, G, group_body, pidx0 + k * C) * 0

    lax.fori_loop(0, NPT // C, chunk_body, 0)

    my_base = half * HALF
    ot_base = (1 - half) * HALF
    pltpu.sync_copy(s.at[pl.ds(ot_base, HALF)],
                    shp.at[pl.ds(sid * HALF, HALF)])
    plsc.subcore_barrier()

    for p0, ln in ((0, C), (C, C), (2 * C, HALF - 2 * C)):
        pltpu.sync_copy(shp.at[pl.ds(psid * HALF + p0, ln)],
                        bufx.at[pl.ds(0, ln)])

        def fin_body(i, _, p0=p0):
            sl = pl.ds(i * 16, 16)
            dst = pl.ds(my_base + p0 + i * 16, 16)
            s[dst] = s[dst] + bufx[sl]
            return 0

        lax.fori_loop(0, ln // 16, fin_body, 0)

    pltpu.sync_copy(s.at[pl.ds(my_base, HALF)],
                    out_hbm.at[pl.ds(view * HW + my_base, HALF)])


@jax.jit
def _render(pts_t, m):
    mesh = plsc.VectorSubcoreMesh(core_axis_name="c", subcore_axis_name="s")
    run = functools.partial(
        pl.kernel,
        out_type=jax.ShapeDtypeStruct((B * HW,), jnp.float32),
        mesh=mesh,
        scratch_types=[
            pltpu.VMEM((HW,), jnp.float32),
            pltpu.VMEM((C,), jnp.float32),
            pltpu.VMEM((C,), jnp.float32),
            pltpu.VMEM((C,), jnp.float32),
            pltpu.VMEM((144,), jnp.float32),
            pltpu.VMEM((32,), jnp.float32),
            pltpu.VMEM_SHARED((16 * HALF,), jnp.float32),
            pltpu.VMEM_SHARED((16 * 32,), jnp.float32),
        ],
        compiler_params=pltpu.CompilerParams(needs_layout_passes=False),
    )(_splat_body)
    return run(pts_t, m)


def kernel(points, azimuth, elevation):
    cos_az, sin_az = jnp.cos(azimuth), jnp.sin(azimuth)
    cos_el, sin_el = jnp.cos(elevation), jnp.sin(elevation)
    z = jnp.zeros_like(cos_az)
    o = jnp.ones_like(cos_az)
    r_az = jnp.stack([
        jnp.stack([cos_az, z, sin_az], axis=-1),
        jnp.stack([z, o, z], axis=-1),
        jnp.stack([-sin_az, z, cos_az], axis=-1),
    ], axis=1)
    r_el = jnp.stack([
        jnp.stack([o, z, z], axis=-1),
        jnp.stack([z, cos_el, -sin_el], axis=-1),
        jnp.stack([z, sin_el, cos_el], axis=-1),
    ], axis=1)
    r = jnp.matmul(r_el, r_az)

    def bf16_round(v):
        u = lax.bitcast_convert_type(v, jnp.uint32)
        rr = (u + 0x7FFF + ((u >> 16) & 1)) & jnp.uint32(0xFFFF0000)
        return lax.bitcast_convert_type(rr, jnp.float32)

    m = bf16_round(r).reshape(B, 9)
    m = jnp.broadcast_to(m[:, :, None], (B, 9, 16)).reshape(-1)
    pts_t = bf16_round(jnp.transpose(points, (0, 2, 1)))
    pts_t = jnp.pad(pts_t, ((0, 0), (0, 0), (0, NP - N))).reshape(-1)
    img = _render(pts_t, m).reshape(B, IMG, IMG)
    return jnp.broadcast_to(img[:, None, :, :], (B, 3, IMG, IMG))


# speedup vs baseline: 48.3380x; 1.0486x over previous
import functools

import jax
import jax.numpy as jnp
from jax import lax
from jax.experimental import pallas as pl
from jax.experimental.pallas import tpu as pltpu
from jax.experimental.pallas import tpu_sc as plsc

IMG = 224
HW = IMG * IMG
N = 100000
B = 16
NP = 100096
C = 12512
G = C // 16
NPT = NP // 2
HALF = HW // 2


def _floor(v):
    t = v.astype(jnp.int32)
    tf = t.astype(jnp.float32)
    return jnp.where(tf > v, t - 1, t)


def _splat_body(pts_hbm, m_hbm, out_hbm, s, bufx, bufy, bufz, rbuf,
                zbuf, shp, shz):
    cid = lax.axis_index("c")
    sid = lax.axis_index("s")
    view = cid * 8 + sid // 2
    half = sid % 2
    psid = sid ^ 1
    pt_base = view * (3 * NP) + half * NPT
    pad_groups = (NP - N) // 16
    nch = NPT // C

    zeros = jnp.zeros((16,), jnp.float32)

    def zbody(i, _):
        s[pl.ds(i * 16, 16)] = zeros
        return 0

    lax.fori_loop(0, HW // 16, zbody, 0)

    pltpu.sync_copy(m_hbm.at[pl.ds(view * 144, 144)], rbuf)
    m = [rbuf[pl.ds(j * 16, 16)] for j in range(9)]

    def zrange_chunk(k, carry):
        base = pt_base + k * C
        pltpu.sync_copy(pts_hbm.at[pl.ds(base, C)], bufx)
        pltpu.sync_copy(pts_hbm.at[pl.ds(base + NP, C)], bufy)
        pltpu.sync_copy(pts_hbm.at[pl.ds(base + 2 * NP, C)], bufz)
        glim = jnp.where((half == 1) & (k == nch - 1), G - pad_groups, G)

        def zrange_group(g, car):
            zmn, zmx = car
            x = bufx[pl.ds(g * 16, 16)]
            y = bufy[pl.ds(g * 16, 16)]
            z = bufz[pl.ds(g * 16, 16)]
            zc = m[6] * x + m[7] * y + m[8] * z
            return jnp.minimum(zmn, zc), jnp.maximum(zmx, zc)

        return lax.fori_loop(0, glim, zrange_group, carry)

    zminv, zmaxv = lax.fori_loop(
        0,
        NPT // C,
        zrange_chunk,
        (jnp.full((16,), jnp.inf), jnp.full((16,), -jnp.inf)),
    )

    zbuf[pl.ds(0, 16)] = zminv
    zbuf[pl.ds(16, 16)] = zmaxv
    pltpu.sync_copy(zbuf, shz.at[pl.ds(sid * 32, 32)])
    plsc.subcore_barrier()
    pltpu.sync_copy(shz.at[pl.ds(psid * 32, 32)], zbuf)
    pzmin = zbuf[pl.ds(0, 16)]
    pzmax = zbuf[pl.ds(16, 16)]

    zmin = jnp.min(jnp.minimum(zminv, pzmin))
    zmax = jnp.max(jnp.maximum(zmaxv, pzmax))
    denom = zmax - zmin + 1e-6
    a_vec = jnp.full((16,), 0.7) / jnp.full((16,), denom)
    c_vec = jnp.full((16,), 0.3) - a_vec * jnp.full((16,), zmin)

    def group_body(g, _):
        x = bufx[pl.ds(g * 16, 16)]
        y = bufy[pl.ds(g * 16, 16)]
        z = bufz[pl.ds(g * 16, 16)]
        px_ = m[0] * x + m[1] * y + m[2] * z + 112.5
        py_ = m[3] * x + m[4] * y + m[5] * z + 112.5
        zc = m[6] * x + m[7] * y + m[8] * z
        feat = a_vec * zc + c_vec
        sxi = px_.astype(jnp.int32)
        syi = py_.astype(jnp.int32)
        fx = px_ - sxi.astype(jnp.float32)
        fy = py_ - syi.astype(jnp.float32)
        gx = 1.0 - fx
        gy = 1.0 - fy
        x1 = sxi - 1
        y1 = syi - 1
        yb2 = syi * IMG
        yb1 = yb2 - IMG
        mask = (x1.astype(jnp.uint32) < IMG - 1) & (
            y1.astype(jnp.uint32) < IMG - 1)
        fgx = feat * gx
        ffx = feat * fx
        plsc.addupdate_scatter(s, [yb1 + x1], fgx * gy, mask=mask)
        plsc.addupdate_scatter(s, [yb2 + x1], fgx * fy, mask=mask)
        plsc.addupdate_scatter(s, [yb1 + sxi], ffx * gy, mask=mask)
        plsc.addupdate_scatter(s, [yb2 + sxi], ffx * fy, mask=mask)
        return 0

    def chunk_body(k, _):
        base = pt_base + k * C
        pltpu.sync_copy(pts_hbm.at[pl.ds(base, C)], bufx)
        pltpu.sync_copy(pts_hbm.at[pl.ds(base + NP, C)], bufy)
        pltpu.sync_copy(pts_hbm.at[pl.ds(base + 2 * NP, C)], bufz)
        glim = jnp.where((half == 1) & (k == nch - 1), G - pad_groups, G)
        return lax.fori_loop(0, glim, group_body, 0)

    lax.fori_loop(0, nch, chunk_body, 0)

    my_base = half * HALF
    ot_base = (1 - half) * HALF
    pltpu.sync_copy(s.at[pl.ds(ot_base, HALF)],
                    shp.at[pl.ds(sid * HALF, HALF)])
    plsc.subcore_barrier()

    for p0, ln in ((0, C), (C, C), (2 * C, HALF - 2 * C)):
        pltpu.sync_copy(shp.at[pl.ds(psid * HALF + p0, ln)],
                        bufx.at[pl.ds(0, ln)])

        def fin_body(i, _, p0=p0):
            sl = pl.ds(i * 16, 16)
            dst = pl.ds(my_base + p0 + i * 16, 16)
            s[dst] = s[dst] + bufx[sl]
            return 0

        lax.fori_loop(0, ln // 16, fin_body, 0)

    for ch in range(3):
        pltpu.sync_copy(
            s.at[pl.ds(my_base, HALF)],
            out_hbm.at[pl.ds((view * 3 + ch) * HW + my_base, HALF)])


@jax.jit
def _render(pts_t, m):
    mesh = plsc.VectorSubcoreMesh(core_axis_name="c", subcore_axis_name="s")
    run = functools.partial(
        pl.kernel,
        out_type=jax.ShapeDtypeStruct((B * 3 * HW,), jnp.float32),
        mesh=mesh,
        scratch_types=[
            pltpu.VMEM((HW,), jnp.float32),
            pltpu.VMEM((C,), jnp.float32),
            pltpu.VMEM((C,), jnp.float32),
            pltpu.VMEM((C,), jnp.float32),
            pltpu.VMEM((144,), jnp.float32),
            pltpu.VMEM((32,), jnp.float32),
            pltpu.VMEM_SHARED((16 * HALF,), jnp.float32),
            pltpu.VMEM_SHARED((16 * 32,), jnp.float32),
        ],
        compiler_params=pltpu.CompilerParams(needs_layout_passes=False),
    )(_splat_body)
    return run(pts_t, m)


def kernel(points, azimuth, elevation):
    cos_az, sin_az = jnp.cos(azimuth), jnp.sin(azimuth)
    cos_el, sin_el = jnp.cos(elevation), jnp.sin(elevation)
    z = jnp.zeros_like(cos_az)
    o = jnp.ones_like(cos_az)
    r_az = jnp.stack([
        jnp.stack([cos_az, z, sin_az], axis=-1),
        jnp.stack([z, o, z], axis=-1),
        jnp.stack([-sin_az, z, cos_az], axis=-1),
    ], axis=1)
    r_el = jnp.stack([
        jnp.stack([o, z, z], axis=-1),
        jnp.stack([z, cos_el, -sin_el], axis=-1),
        jnp.stack([z, sin_el, cos_el], axis=-1),
    ], axis=1)
    r = jnp.matmul(r_el, r_az)

    def bf16_round(v):
        u = lax.bitcast_convert_type(v, jnp.uint32)
        rr = (u + 0x7FFF + ((u >> 16) & 1)) & jnp.uint32(0xFFFF0000)
        return lax.bitcast_convert_type(rr, jnp.float32)

    scale = jnp.array([112.0, 112.0, 1.0], jnp.float32)[None, :, None]
    m = (bf16_round(r) * scale).reshape(B, 9)
    m = jnp.broadcast_to(m[:, :, None], (B, 9, 16)).reshape(-1)
    pts_t = bf16_round(jnp.transpose(points, (0, 2, 1)))
    pts_t = jnp.pad(pts_t, ((0, 0), (0, 0), (0, NP - N))).reshape(-1)
    return _render(pts_t, m).reshape(B, 3, IMG, IMG)


# speedup vs baseline: 49.2237x; 1.0183x over previous
import functools

import jax
import jax.numpy as jnp
from jax import lax
from jax.experimental import pallas as pl
from jax.experimental.pallas import tpu as pltpu
from jax.experimental.pallas import tpu_sc as plsc

IMG = 224
HW = IMG * IMG
N = 100000
B = 16
NP = 100096
C = 12512
G = C // 16
NPT = NP // 2
HALF = HW // 2


def _floor(v):
    t = v.astype(jnp.int32)
    tf = t.astype(jnp.float32)
    return jnp.where(tf > v, t - 1, t)


def _splat_body(pts_hbm, m_hbm, out_hbm, s, bufx, bufy, bufz, rbuf,
                zbuf, shp, shz):
    cid = lax.axis_index("c")
    sid = lax.axis_index("s")
    view = cid * 8 + sid // 2
    half = sid % 2
    psid = sid ^ 1
    pt_base = view * (3 * NP) + half * NPT
    pad_groups = (NP - N) // 16
    nch = NPT // C

    zeros = jnp.zeros((16,), jnp.float32)

    def zbody(i, _):
        s[pl.ds(i * 16, 16)] = zeros
        return 0

    lax.fori_loop(0, HW // 16, zbody, 0)

    pltpu.sync_copy(m_hbm.at[pl.ds(view * 144, 144)], rbuf)
    m = [rbuf[pl.ds(j * 16, 16)] for j in range(9)]

    def zrange_chunk(k, carry):
        base = pt_base + k * C
        pltpu.sync_copy(pts_hbm.at[pl.ds(base, C)], bufx)
        pltpu.sync_copy(pts_hbm.at[pl.ds(base + NP, C)], bufy)
        pltpu.sync_copy(pts_hbm.at[pl.ds(base + 2 * NP, C)], bufz)
        glim = jnp.where((half == 1) & (k == nch - 1),
                         G // 2 - pad_groups // 2, G // 2)

        def zrange_group(g, car):
            zmn, zmx = car
            for u in range(2):
                o = g * 32 + u * 16
                x = bufx[pl.ds(o, 16)]
                y = bufy[pl.ds(o, 16)]
                z = bufz[pl.ds(o, 16)]
                zc = m[6] * x + m[7] * y + m[8] * z
                zmn = jnp.minimum(zmn, zc)
                zmx = jnp.maximum(zmx, zc)
            return zmn, zmx

        return lax.fori_loop(0, glim, zrange_group, carry)

    zminv, zmaxv = lax.fori_loop(
        0,
        NPT // C,
        zrange_chunk,
        (jnp.full((16,), jnp.inf), jnp.full((16,), -jnp.inf)),
    )

    zbuf[pl.ds(0, 16)] = zminv
    zbuf[pl.ds(16, 16)] = zmaxv
    pltpu.sync_copy(zbuf, shz.at[pl.ds(sid * 32, 32)])
    plsc.subcore_barrier()
    pltpu.sync_copy(shz.at[pl.ds(psid * 32, 32)], zbuf)
    pzmin = zbuf[pl.ds(0, 16)]
    pzmax = zbuf[pl.ds(16, 16)]

    zmin = jnp.min(jnp.minimum(zminv, pzmin))
    zmax = jnp.max(jnp.maximum(zmaxv, pzmax))
    denom = zmax - zmin + 1e-6
    a_vec = jnp.full((16,), 0.7) / jnp.full((16,), denom)
    c_vec = jnp.full((16,), 0.3) - a_vec * jnp.full((16,), zmin)

    def group_body(g, _):
        for u in range(2):
            o = g * 32 + u * 16
            x = bufx[pl.ds(o, 16)]
            y = bufy[pl.ds(o, 16)]
            z = bufz[pl.ds(o, 16)]
            px_ = m[0] * x + m[1] * y + m[2] * z + 112.5
            py_ = m[3] * x + m[4] * y + m[5] * z + 112.5
            zc = m[6] * x + m[7] * y + m[8] * z
            feat = a_vec * zc + c_vec
            sxi = px_.astype(jnp.int32)
            syi = py_.astype(jnp.int32)
            fx = px_ - sxi.astype(jnp.float32)
            fy = py_ - syi.astype(jnp.float32)
            gx = 1.0 - fx
            gy = 1.0 - fy
            x1 = sxi - 1
            y1 = syi - 1
            yb2 = syi * IMG
            yb1 = yb2 - IMG
            mask = (x1.astype(jnp.uint32) < IMG - 1) & (
                y1.astype(jnp.uint32) < IMG - 1)
            fgx = feat * gx
            ffx = feat * fx
            plsc.addupdate_scatter(s, [yb1 + x1], fgx * gy, mask=mask)
            plsc.addupdate_scatter(s, [yb2 + x1], fgx * fy, mask=mask)
            plsc.addupdate_scatter(s, [yb1 + sxi], ffx * gy, mask=mask)
            plsc.addupdate_scatter(s, [yb2 + sxi], ffx * fy, mask=mask)
        return 0

    def chunk_body(k, _):
        base = pt_base + k * C
        pltpu.sync_copy(pts_hbm.at[pl.ds(base, C)], bufx)
        pltpu.sync_copy(pts_hbm.at[pl.ds(base + NP, C)], bufy)
        pltpu.sync_copy(pts_hbm.at[pl.ds(base + 2 * NP, C)], bufz)
        glim = jnp.where((half == 1) & (k == nch - 1),
                         G // 2 - pad_groups // 2, G // 2)
        return lax.fori_loop(0, glim, group_body, 0)

    lax.fori_loop(0, nch, chunk_body, 0)

    my_base = half * HALF
    ot_base = (1 - half) * HALF
    pltpu.sync_copy(s.at[pl.ds(ot_base, HALF)],
                    shp.at[pl.ds(sid * HALF, HALF)])
    plsc.subcore_barrier()

    for p0, ln in ((0, C), (C, C), (2 * C, HALF - 2 * C)):
        pltpu.sync_copy(shp.at[pl.ds(psid * HALF + p0, ln)],
                        bufx.at[pl.ds(0, ln)])

        def fin_body(i, _, p0=p0):
            sl = pl.ds(i * 16, 16)
            dst = pl.ds(my_base + p0 + i * 16, 16)
            s[dst] = s[dst] + bufx[sl]
            return 0

        lax.fori_loop(0, ln // 16, fin_body, 0)

    for ch in range(3):
        pltpu.sync_copy(
            s.at[pl.ds(my_base, HALF)],
            out_hbm.at[pl.ds((view * 3 + ch) * HW + my_base, HALF)])


@jax.jit
def _render(pts_t, m):
    mesh = plsc.VectorSubcoreMesh(core_axis_name="c", subcore_axis_name="s")
    run = functools.partial(
        pl.kernel,
        out_type=jax.ShapeDtypeStruct((B * 3 * HW,), jnp.float32),
        mesh=mesh,
        scratch_types=[
            pltpu.VMEM((HW,), jnp.float32),
            pltpu.VMEM((C,), jnp.float32),
            pltpu.VMEM((C,), jnp.float32),
            pltpu.VMEM((C,), jnp.float32),
            pltpu.VMEM((144,), jnp.float32),
            pltpu.VMEM((32,), jnp.float32),
            pltpu.VMEM_SHARED((16 * HALF,), jnp.float32),
            pltpu.VMEM_SHARED((16 * 32,), jnp.float32),
        ],
        compiler_params=pltpu.CompilerParams(needs_layout_passes=False),
    )(_splat_body)
    return run(pts_t, m)


def kernel(points, azimuth, elevation):
    cos_az, sin_az = jnp.cos(azimuth), jnp.sin(azimuth)
    cos_el, sin_el = jnp.cos(elevation), jnp.sin(elevation)
    z = jnp.zeros_like(cos_az)
    o = jnp.ones_like(cos_az)
    r_az = jnp.stack([
        jnp.stack([cos_az, z, sin_az], axis=-1),
        jnp.stack([z, o, z], axis=-1),
        jnp.stack([-sin_az, z, cos_az], axis=-1),
    ], axis=1)
    r_el = jnp.stack([
        jnp.stack([o, z, z], axis=-1),
        jnp.stack([z, cos_el, -sin_el], axis=-1),
        jnp.stack([z, sin_el, cos_el], axis=-1),
    ], axis=1)
    r = jnp.matmul(r_el, r_az)

    def bf16_round(v):
        u = lax.bitcast_convert_type(v, jnp.uint32)
        rr = (u + 0x7FFF + ((u >> 16) & 1)) & jnp.uint32(0xFFFF0000)
        return lax.bitcast_convert_type(rr, jnp.float32)

    scale = jnp.array([112.0, 112.0, 1.0], jnp.float32)[None, :, None]
    m = (bf16_round(r) * scale).reshape(B, 9)
    m = jnp.broadcast_to(m[:, :, None], (B, 9, 16)).reshape(-1)
    pts_t = bf16_round(jnp.transpose(points, (0, 2, 1)))
    pts_t = jnp.pad(pts_t, ((0, 0), (0, 0), (0, NP - N))).reshape(-1)
    return _render(pts_t, m).reshape(B, 3, IMG, IMG)


# speedup vs baseline: 51.6133x; 1.0485x over previous
import functools

import jax
import jax.numpy as jnp
from jax import lax
from jax.experimental import pallas as pl
from jax.experimental.pallas import tpu as pltpu
from jax.experimental.pallas import tpu_sc as plsc

IMG = 224
HW = IMG * IMG
N = 100000
B = 16
NP = 100096
C = 12512
G = C // 16
NPT = NP // 2
HALF = HW // 2


def _floor(v):
    t = v.astype(jnp.int32)
    tf = t.astype(jnp.float32)
    return jnp.where(tf > v, t - 1, t)


def _splat_body(pts_hbm, m_hbm, out_hbm, s, bufx, bufy, bufz, rbuf,
                zbuf, shp, shz):
    cid = lax.axis_index("c")
    sid = lax.axis_index("s")
    view = cid * 8 + sid // 2
    half = sid % 2
    psid = sid ^ 1
    pt_base = view * (3 * NP) + half * NPT
    pad_groups = (NP - N) // 16
    nch = NPT // C

    zeros = jnp.zeros((16,), jnp.float32)

    def zbody(i, _):
        for u in range(4):
            s[pl.ds(i * 64 + u * 16, 16)] = zeros
        return 0

    lax.fori_loop(0, HW // 64, zbody, 0)

    pltpu.sync_copy(m_hbm.at[pl.ds(view * 144, 144)], rbuf)
    m = [rbuf[pl.ds(j * 16, 16)] for j in range(9)]

    def zrange_chunk(k, carry):
        base = pt_base + k * C
        pltpu.sync_copy(pts_hbm.at[pl.ds(base, C)], bufx)
        pltpu.sync_copy(pts_hbm.at[pl.ds(base + NP, C)], bufy)
        pltpu.sync_copy(pts_hbm.at[pl.ds(base + 2 * NP, C)], bufz)
        glim = jnp.where((half == 1) & (k == nch - 1),
                         G // 2 - pad_groups // 2, G // 2)

        def zrange_group(g, car):
            zmn, zmx = car
            for u in range(2):
                o = g * 32 + u * 16
                x = bufx[pl.ds(o, 16)]
                y = bufy[pl.ds(o, 16)]
                z = bufz[pl.ds(o, 16)]
                zc = m[6] * x + m[7] * y + m[8] * z
                zmn = jnp.minimum(zmn, zc)
                zmx = jnp.maximum(zmx, zc)
            return zmn, zmx

        return lax.fori_loop(0, glim, zrange_group, carry)

    zminv, zmaxv = lax.fori_loop(
        0,
        NPT // C,
        zrange_chunk,
        (jnp.full((16,), jnp.inf), jnp.full((16,), -jnp.inf)),
    )

    zbuf[pl.ds(0, 16)] = zminv
    zbuf[pl.ds(16, 16)] = zmaxv
    pltpu.sync_copy(zbuf, shz.at[pl.ds(sid * 32, 32)])
    plsc.subcore_barrier()
    pltpu.sync_copy(shz.at[pl.ds(psid * 32, 32)], zbuf)
    pzmin = zbuf[pl.ds(0, 16)]
    pzmax = zbuf[pl.ds(16, 16)]

    zmin = jnp.min(jnp.minimum(zminv, pzmin))
    zmax = jnp.max(jnp.maximum(zmaxv, pzmax))
    denom = zmax - zmin + 1e-6
    a_vec = jnp.full((16,), 0.7) / jnp.full((16,), denom)
    c_vec = jnp.full((16,), 0.3) - a_vec * jnp.full((16,), zmin)

    def group_body(g, _):
        for u in range(2):
            o = g * 32 + u * 16
            x = bufx[pl.ds(o, 16)]
            y = bufy[pl.ds(o, 16)]
            z = bufz[pl.ds(o, 16)]
            px_ = m[0] * x + m[1] * y + m[2] * z + 112.5
            py_ = m[3] * x + m[4] * y + m[5] * z + 112.5
            zc = m[6] * x + m[7] * y + m[8] * z
            feat = a_vec * zc + c_vec
            sxi = px_.astype(jnp.int32)
            syi = py_.astype(jnp.int32)
            fx = px_ - sxi.astype(jnp.float32)
            fy = py_ - syi.astype(jnp.float32)
            gx = 1.0 - fx
            gy = 1.0 - fy
            x1 = sxi - 1
            y1 = syi - 1
            yb2 = syi * IMG
            yb1 = yb2 - IMG
            mask = (x1.astype(jnp.uint32) < IMG - 1) & (
                y1.astype(jnp.uint32) < IMG - 1)
            fgx = feat * gx
            ffx = feat * fx
            plsc.addupdate_scatter(s, [yb1 + x1], fgx * gy, mask=mask)
            plsc.addupdate_scatter(s, [yb2 + x1], fgx * fy, mask=mask)
            plsc.addupdate_scatter(s, [yb1 + sxi], ffx * gy, mask=mask)
            plsc.addupdate_scatter(s, [yb2 + sxi], ffx * fy, mask=mask)
        return 0

    def chunk_body(k, _):
        base = pt_base + k * C
        pltpu.sync_copy(pts_hbm.at[pl.ds(base, C)], bufx)
        pltpu.sync_copy(pts_hbm.at[pl.ds(base + NP, C)], bufy)
        pltpu.sync_copy(pts_hbm.at[pl.ds(base + 2 * NP, C)], bufz)
        glim = jnp.where((half == 1) & (k == nch - 1),
                         G // 2 - pad_groups // 2, G // 2)
        return lax.fori_loop(0, glim, group_body, 0)

    lax.fori_loop(0, nch, chunk_body, 0)

    my_base = half * HALF
    ot_base = (1 - half) * HALF
    pltpu.sync_copy(s.at[pl.ds(ot_base, HALF)],
                    shp.at[pl.ds(sid * HALF, HALF)])
    plsc.subcore_barrier()

    for p0, ln in ((0, 12480), (12480, 12480), (24960, HALF - 24960)):
        pltpu.sync_copy(shp.at[pl.ds(psid * HALF + p0, ln)],
                        bufx.at[pl.ds(0, ln)])

        def fin_body(i, _, p0=p0):
            for u in range(4):
                sl = pl.ds(i * 64 + u * 16, 16)
                dst = pl.ds(my_base + p0 + i * 64 + u * 16, 16)
                s[dst] = s[dst] + bufx[sl]
            return 0

        lax.fori_loop(0, ln // 64, fin_body, 0)

    for ch in range(3):
        pltpu.sync_copy(
            s.at[pl.ds(my_base, HALF)],
            out_hbm.at[pl.ds((view * 3 + ch) * HW + my_base, HALF)])


@jax.jit
def _render(pts_t, m):
    mesh = plsc.VectorSubcoreMesh(core_axis_name="c", subcore_axis_name="s")
    run = functools.partial(
        pl.kernel,
        out_type=jax.ShapeDtypeStruct((B * 3 * HW,), jnp.float32),
        mesh=mesh,
        scratch_types=[
            pltpu.VMEM((HW,), jnp.float32),
            pltpu.VMEM((C,), jnp.float32),
            pltpu.VMEM((C,), jnp.float32),
            pltpu.VMEM((C,), jnp.float32),
            pltpu.VMEM((144,), jnp.float32),
            pltpu.VMEM((32,), jnp.float32),
            pltpu.VMEM_SHARED((16 * HALF,), jnp.float32),
            pltpu.VMEM_SHARED((16 * 32,), jnp.float32),
        ],
        compiler_params=pltpu.CompilerParams(needs_layout_passes=False),
    )(_splat_body)
    return run(pts_t, m)


def kernel(points, azimuth, elevation):
    cos_az, sin_az = jnp.cos(azimuth), jnp.sin(azimuth)
    cos_el, sin_el = jnp.cos(elevation), jnp.sin(elevation)
    z = jnp.zeros_like(cos_az)
    o = jnp.ones_like(cos_az)
    r_az = jnp.stack([
        jnp.stack([cos_az, z, sin_az], axis=-1),
        jnp.stack([z, o, z], axis=-1),
        jnp.stack([-sin_az, z, cos_az], axis=-1),
    ], axis=1)
    r_el = jnp.stack([
        jnp.stack([o, z, z], axis=-1),
        jnp.stack([z, cos_el, -sin_el], axis=-1),
        jnp.stack([z, sin_el, cos_el], axis=-1),
    ], axis=1)
    r = jnp.matmul(r_el, r_az)

    def bf16_round(v):
        u = lax.bitcast_convert_type(v, jnp.uint32)
        rr = (u + 0x7FFF + ((u >> 16) & 1)) & jnp.uint32(0xFFFF0000)
        return lax.bitcast_convert_type(rr, jnp.float32)

    scale = jnp.array([112.0, 112.0, 1.0], jnp.float32)[None, :, None]
    m = (bf16_round(r) * scale).reshape(B, 9)
    m = jnp.broadcast_to(m[:, :, None], (B, 9, 16)).reshape(-1)
    pts_t = bf16_round(jnp.transpose(points, (0, 2, 1)))
    pts_t = jnp.pad(pts_t, ((0, 0), (0, 0), (0, NP - N))).reshape(-1)
    return _render(pts_t, m).reshape(B, 3, IMG, IMG)


# speedup vs baseline: 54.3315x; 1.0527x over previous
import functools

import jax
import jax.numpy as jnp
from jax import lax
from jax.experimental import pallas as pl
from jax.experimental.pallas import tpu as pltpu
from jax.experimental.pallas import tpu_sc as plsc

IMG = 224
HW = IMG * IMG
N = 100000
B = 16
NP = 100096
C = 2176
G2 = C // 32
NPT = NP // 2
NCH = NPT // C
HALF = HW // 2


def _floor(v):
    t = v.astype(jnp.int32)
    tf = t.astype(jnp.float32)
    return jnp.where(tf > v, t - 1, t)


def _splat_body(pts_hbm, m_hbm, out_hbm, s, bxa, bya, bza, bxb, byb, bzb,
                rbuf, zbuf, cmb, shp, shz, sema, semb):
    cid = lax.axis_index("c")
    sid = lax.axis_index("s")
    view = cid * 8 + sid // 2
    half = sid % 2
    psid = sid ^ 1
    pt_base = view * (3 * NP) + half * NPT
    pad_dgroups = (NP - N) // 32
    bufsets = ((bxa, bya, bza, sema), (bxb, byb, bzb, semb))

    def issue(k, par):
        bx, by, bz, sem = bufsets[par]
        base = pt_base + k * C
        return [
            pltpu.async_copy(pts_hbm.at[pl.ds(base, C)], bx, sem),
            pltpu.async_copy(pts_hbm.at[pl.ds(base + NP, C)], by, sem),
            pltpu.async_copy(pts_hbm.at[pl.ds(base + 2 * NP, C)], bz, sem),
        ]

    zeros = jnp.zeros((16,), jnp.float32)

    def zbody(i, _):
        for u in range(4):
            s[pl.ds(i * 64 + u * 16, 16)] = zeros
        return 0

    lax.fori_loop(0, HW // 64, zbody, 0)

    pltpu.sync_copy(m_hbm.at[pl.ds(view * 144, 144)], rbuf)
    m = [rbuf[pl.ds(j * 16, 16)] for j in range(9)]

    def zrange_group(bx, by, bz):
        def body(g, car):
            zmn, zmx = car
            for u in range(2):
                o = g * 32 + u * 16
                x = bx[pl.ds(o, 16)]
                y = by[pl.ds(o, 16)]
                z = bz[pl.ds(o, 16)]
                zc = m[6] * x + m[7] * y + m[8] * z
                zmn = jnp.minimum(zmn, zc)
                zmx = jnp.maximum(zmx, zc)
            return zmn, zmx
        return body

    carry = (jnp.full((16,), jnp.inf), jnp.full((16,), -jnp.inf))
    h = issue(0, 0)
    for k in range(NCH):
        hn = issue(k + 1, (k + 1) % 2) if k + 1 < NCH else None
        for hh in h:
            hh.wait()
        bx, by, bz, _ = bufsets[k % 2]
        if k == NCH - 1:
            glim = jnp.where(half == 1, G2 - pad_dgroups, G2)
        else:
            glim = G2
        carry = lax.fori_loop(0, glim, zrange_group(bx, by, bz), carry)
        h = hn
    zminv, zmaxv = carry

    zbuf[pl.ds(0, 16)] = zminv
    zbuf[pl.ds(16, 16)] = zmaxv
    pltpu.sync_copy(zbuf, shz.at[pl.ds(sid * 32, 32)])
    plsc.subcore_barrier()
    pltpu.sync_copy(shz.at[pl.ds(psid * 32, 32)], zbuf)
    pzmin = zbuf[pl.ds(0, 16)]
    pzmax = zbuf[pl.ds(16, 16)]

    zmin = jnp.min(jnp.minimum(zminv, pzmin))
    zmax = jnp.max(jnp.maximum(zmaxv, pzmax))
    denom = zmax - zmin + 1e-6
    a_vec = jnp.full((16,), 0.7) / jnp.full((16,), denom)
    c_vec = jnp.full((16,), 0.3) - a_vec * jnp.full((16,), zmin)

    def group_body(bx, by, bz):
      def body(g, _):
        for u in range(2):
            o = g * 32 + u * 16
            x = bx[pl.ds(o, 16)]
            y = by[pl.ds(o, 16)]
            z = bz[pl.ds(o, 16)]
            px_ = m[0] * x + m[1] * y + m[2] * z + 112.5
            py_ = m[3] * x + m[4] * y + m[5] * z + 112.5
            zc = m[6] * x + m[7] * y + m[8] * z
            feat = a_vec * zc + c_vec
            sxi = px_.astype(jnp.int32)
            syi = py_.astype(jnp.int32)
            fx = px_ - sxi.astype(jnp.float32)
            fy = py_ - syi.astype(jnp.float32)
            gx = 1.0 - fx
            gy = 1.0 - fy
            x1 = sxi - 1
            y1 = syi - 1
            yb2 = syi * IMG
            yb1 = yb2 - IMG
            mask = (x1.astype(jnp.uint32) < IMG - 1) & (
                y1.astype(jnp.uint32) < IMG - 1)
            fgx = feat * gx
            ffx = feat * fx
            plsc.addupdate_scatter(s, [yb1 + x1], fgx * gy, mask=mask)
            plsc.addupdate_scatter(s, [yb2 + x1], fgx * fy, mask=mask)
            plsc.addupdate_scatter(s, [yb1 + sxi], ffx * gy, mask=mask)
            plsc.addupdate_scatter(s, [yb2 + sxi], ffx * fy, mask=mask)
        return 0
      return body

    h = issue(0, 0)
    for k in range(NCH):
        hn = issue(k + 1, (k + 1) % 2) if k + 1 < NCH else None
        for hh in h:
            hh.wait()
        bx, by, bz, _ = bufsets[k % 2]
        if k == NCH - 1:
            glim = jnp.where(half == 1, G2 - pad_dgroups, G2)
        else:
            glim = G2
        lax.fori_loop(0, glim, group_body(bx, by, bz), 0)
        h = hn

    my_base = half * HALF
    ot_base = (1 - half) * HALF
    pltpu.sync_copy(s.at[pl.ds(ot_base, HALF)],
                    shp.at[pl.ds(sid * HALF, HALF)])
    plsc.subcore_barrier()

    pltpu.sync_copy(shp.at[pl.ds(psid * HALF, HALF)], cmb)

    def fin_body(i, _):
        for u in range(4):
            sl = pl.ds(i * 64 + u * 16, 16)
            dst = pl.ds(my_base + i * 64 + u * 16, 16)
            s[dst] = s[dst] + cmb[sl]
        return 0

    lax.fori_loop(0, HALF // 64, fin_body, 0)

    for ch in range(3):
        pltpu.sync_copy(
            s.at[pl.ds(my_base, HALF)],
            out_hbm.at[pl.ds((view * 3 + ch) * HW + my_base, HALF)])


@jax.jit
def _render(pts_t, m):
    mesh = plsc.VectorSubcoreMesh(core_axis_name="c", subcore_axis_name="s")
    run = functools.partial(
        pl.kernel,
        out_type=jax.ShapeDtypeStruct((B * 3 * HW,), jnp.float32),
        mesh=mesh,
        scratch_types=[
            pltpu.VMEM((HW,), jnp.float32),
            pltpu.VMEM((C,), jnp.float32),
            pltpu.VMEM((C,), jnp.float32),
            pltpu.VMEM((C,), jnp.float32),
            pltpu.VMEM((C,), jnp.float32),
            pltpu.VMEM((C,), jnp.float32),
            pltpu.VMEM((C,), jnp.float32),
            pltpu.VMEM((144,), jnp.float32),
            pltpu.VMEM((32,), jnp.float32),
            pltpu.VMEM((HALF,), jnp.float32),
            pltpu.VMEM_SHARED((16 * HALF,), jnp.float32),
            pltpu.VMEM_SHARED((16 * 32,), jnp.float32),
            pltpu.SemaphoreType.DMA,
            pltpu.SemaphoreType.DMA,
        ],
        compiler_params=pltpu.CompilerParams(needs_layout_passes=False),
    )(_splat_body)
    return run(pts_t, m)


def kernel(points, azimuth, elevation):
    cos_az, sin_az = jnp.cos(azimuth), jnp.sin(azimuth)
    cos_el, sin_el = jnp.cos(elevation), jnp.sin(elevation)
    z = jnp.zeros_like(cos_az)
    o = jnp.ones_like(cos_az)
    r_az = jnp.stack([
        jnp.stack([cos_az, z, sin_az], axis=-1),
        jnp.stack([z, o, z], axis=-1),
        jnp.stack([-sin_az, z, cos_az], axis=-1),
    ], axis=1)
    r_el = jnp.stack([
        jnp.stack([o, z, z], axis=-1),
        jnp.stack([z, cos_el, -sin_el], axis=-1),
        jnp.stack([z, sin_el, cos_el], axis=-1),
    ], axis=1)
    r = jnp.matmul(r_el, r_az)

    def bf16_round(v):
        u = lax.bitcast_convert_type(v, jnp.uint32)
        rr = (u + 0x7FFF + ((u >> 16) & 1)) & jnp.uint32(0xFFFF0000)
        return lax.bitcast_convert_type(rr, jnp.float32)

    scale = jnp.array([112.0, 112.0, 1.0], jnp.float32)[None, :, None]
    m = (bf16_round(r) * scale).reshape(B, 9)
    m = jnp.broadcast_to(m[:, :, None], (B, 9, 16)).reshape(-1)
    pts_t = bf16_round(jnp.transpose(points, (0, 2, 1)))
    pts_t = jnp.pad(pts_t, ((0, 0), (0, 0), (0, NP - N))).reshape(-1)
    return _render(pts_t, m).reshape(B, 3, IMG, IMG)


# speedup vs baseline: 54.4078x; 1.0014x over previous
import functools

import jax
import jax.numpy as jnp
from jax import lax
from jax.experimental import pallas as pl
from jax.experimental.pallas import tpu as pltpu
from jax.experimental.pallas import tpu_sc as plsc

IMG = 224
HW = IMG * IMG
N = 100000
B = 16
NP = 100096
C = 2176
G2 = C // 32
NPT = NP // 2
NCH = NPT // C
HALF = HW // 2


def _splat_body(pts_hbm, m_hbm, out_hbm, s, bxa, bya, bza, bxb, byb, bzb,
                rbuf, zbuf, cmb, shp, shz, sema, semb):
    cid = lax.axis_index("c")
    sid = lax.axis_index("s")
    view = cid * 8 + sid // 2
    half = sid % 2
    psid = sid ^ 1
    pt_base = view * (3 * NP) + half * NPT
    pad_dgroups = (NP - N) // 32
    bufsets = ((bxa, bya, bza, sema), (bxb, byb, bzb, semb))

    def issue(k, par):
        bx, by, bz, sem = bufsets[par]
        base = pt_base + k * C
        return [
            pltpu.async_copy(pts_hbm.at[pl.ds(base, C)], bx, sem),
            pltpu.async_copy(pts_hbm.at[pl.ds(base + NP, C)], by, sem),
            pltpu.async_copy(pts_hbm.at[pl.ds(base + 2 * NP, C)], bz, sem),
        ]

    zeros = jnp.zeros((16,), jnp.float32)

    def zbody(i, _):
        for u in range(4):
            s[pl.ds(i * 64 + u * 16, 16)] = zeros
        return 0

    lax.fori_loop(0, HW // 64, zbody, 0)

    pltpu.sync_copy(m_hbm.at[pl.ds(view * 144, 144)], rbuf)
    m = [rbuf[pl.ds(j * 16, 16)] for j in range(9)]

    def zrange_group(bx, by, bz):
        def body(g, car):
            zmn, zmx = car
            for u in range(2):
                o = g * 32 + u * 16
                x = bx[pl.ds(o, 16)]
                y = by[pl.ds(o, 16)]
                z = bz[pl.ds(o, 16)]
                zc = m[6] * x + m[7] * y + m[8] * z
                zmn = jnp.minimum(zmn, zc)
                zmx = jnp.maximum(zmx, zc)
            return zmn, zmx
        return body

    carry = (jnp.full((16,), jnp.inf), jnp.full((16,), -jnp.inf))
    h = issue(0, 0)
    for k in range(NCH):
        hn = issue(k + 1, (k + 1) % 2) if k + 1 < NCH else None
        for hh in h:
            hh.wait()
        bx, by, bz, _ = bufsets[k % 2]
        if k == NCH - 1:
            glim = jnp.where(half == 1, G2 - pad_dgroups, G2)
        else:
            glim = G2
        carry = lax.fori_loop(0, glim, zrange_group(bx, by, bz), carry)
        h = hn
    zminv, zmaxv = carry

    zbuf[pl.ds(0, 16)] = zminv
    zbuf[pl.ds(16, 16)] = zmaxv
    pltpu.sync_copy(zbuf, shz.at[pl.ds(sid * 32, 32)])
    plsc.subcore_barrier()
    pltpu.sync_copy(shz.at[pl.ds(psid * 32, 32)], zbuf)
    pzmin = zbuf[pl.ds(0, 16)]
    pzmax = zbuf[pl.ds(16, 16)]

    zmin = jnp.min(jnp.minimum(zminv, pzmin))
    zmax = jnp.max(jnp.maximum(zmaxv, pzmax))
    denom = zmax - zmin + 1e-6
    a_vec = jnp.full((16,), 0.7) / jnp.full((16,), denom)
    c_vec = jnp.full((16,), 0.3) - a_vec * jnp.full((16,), zmin)

    def group_body(bx, by, bz):
      def body(g, _):
        for u in range(2):
            o = g * 32 + u * 16
            x = bx[pl.ds(o, 16)]
            y = by[pl.ds(o, 16)]
            z = bz[pl.ds(o, 16)]
            px_ = m[0] * x + m[1] * y + m[2] * z + 112.5
            py_ = m[3] * x + m[4] * y + m[5] * z + 112.5
            zc = m[6] * x + m[7] * y + m[8] * z
            feat = a_vec * zc + c_vec
            sxi = px_.astype(jnp.int32)
            syi = py_.astype(jnp.int32)
            fx = px_ - sxi.astype(jnp.float32)
            fy = py_ - syi.astype(jnp.float32)
            gx = 1.0 - fx
            gy = 1.0 - fy
            x1 = sxi - 1
            y1 = syi - 1
            yb2 = syi * IMG
            yb1 = yb2 - IMG
            mask = (x1.astype(jnp.uint32) < IMG - 1) & (
                y1.astype(jnp.uint32) < IMG - 1)
            fgx = feat * gx
            ffx = feat * fx
            plsc.addupdate_scatter(s, [yb1 + x1], fgx * gy, mask=mask)
            plsc.addupdate_scatter(s, [yb2 + x1], fgx * fy, mask=mask)
            plsc.addupdate_scatter(s, [yb1 + sxi], ffx * gy, mask=mask)
            plsc.addupdate_scatter(s, [yb2 + sxi], ffx * fy, mask=mask)
        return 0
      return body

    h = issue(0, 0)
    for k in range(NCH):
        hn = issue(k + 1, (k + 1) % 2) if k + 1 < NCH else None
        for hh in h:
            hh.wait()
        bx, by, bz, _ = bufsets[k % 2]
        if k == NCH - 1:
            glim = jnp.where(half == 1, G2 - pad_dgroups, G2)
        else:
            glim = G2
        lax.fori_loop(0, glim, group_body(bx, by, bz), 0)
        h = hn

    my_base = half * HALF
    ot_base = (1 - half) * HALF
    pltpu.sync_copy(s.at[pl.ds(ot_base, HALF)],
                    shp.at[pl.ds(sid * HALF, HALF)])
    plsc.subcore_barrier()

    pltpu.sync_copy(shp.at[pl.ds(psid * HALF, HALF)], cmb)

    def fin_body(i, _):
        for u in range(4):
            sl = pl.ds(i * 64 + u * 16, 16)
            dst = pl.ds(my_base + i * 64 + u * 16, 16)
            s[dst] = s[dst] + cmb[sl]
        return 0

    lax.fori_loop(0, HALF // 64, fin_body, 0)

    for ch in range(3):
        pltpu.sync_copy(
            s.at[pl.ds(my_base, HALF)],
            out_hbm.at[pl.ds((view * 3 + ch) * HW + my_base, HALF)])


@jax.jit
def _render(pts_t, m):
    mesh = plsc.VectorSubcoreMesh(core_axis_name="c", subcore_axis_name="s")
    run = functools.partial(
        pl.kernel,
        out_type=jax.ShapeDtypeStruct((B * 3 * HW,), jnp.float32),
        mesh=mesh,
        scratch_types=[
            pltpu.VMEM((HW,), jnp.float32),
            pltpu.VMEM((C,), jnp.float32),
            pltpu.VMEM((C,), jnp.float32),
            pltpu.VMEM((C,), jnp.float32),
            pltpu.VMEM((C,), jnp.float32),
            pltpu.VMEM((C,), jnp.float32),
            pltpu.VMEM((C,), jnp.float32),
            pltpu.VMEM((144,), jnp.float32),
            pltpu.VMEM((32,), jnp.float32),
            pltpu.VMEM((HALF,), jnp.float32),
            pltpu.VMEM_SHARED((16 * HALF,), jnp.float32),
            pltpu.VMEM_SHARED((16 * 32,), jnp.float32),
            pltpu.SemaphoreType.DMA,
            pltpu.SemaphoreType.DMA,
        ],
        compiler_params=pltpu.CompilerParams(needs_layout_passes=False),
    )(_splat_body)
    return run(pts_t, m)


def kernel(points, azimuth, elevation):
    cos_az, sin_az = jnp.cos(azimuth), jnp.sin(azimuth)
    cos_el, sin_el = jnp.cos(elevation), jnp.sin(elevation)
    z = jnp.zeros_like(cos_az)
    o = jnp.ones_like(cos_az)
    r_az = jnp.stack([
        jnp.stack([cos_az, z, sin_az], axis=-1),
        jnp.stack([z, o, z], axis=-1),
        jnp.stack([-sin_az, z, cos_az], axis=-1),
    ], axis=1)
    r_el = jnp.stack([
        jnp.stack([o, z, z], axis=-1),
        jnp.stack([z, cos_el, -sin_el], axis=-1),
        jnp.stack([z, sin_el, cos_el], axis=-1),
    ], axis=1)
    r = jnp.matmul(r_el, r_az)

    def bf16_round(v):
        u = lax.bitcast_convert_type(v, jnp.uint32)
        rr = (u + 0x7FFF + ((u >> 16) & 1)) & jnp.uint32(0xFFFF0000)
        return lax.bitcast_convert_type(rr, jnp.float32)

    scale = jnp.array([112.0, 112.0, 1.0], jnp.float32)[None, :, None]
    m = (bf16_round(r) * scale).reshape(B, 9)
    m = jnp.broadcast_to(m[:, :, None], (B, 9, 16)).reshape(-1)
    pts_t = bf16_round(jnp.transpose(points, (0, 2, 1)))
    pts_t = jnp.pad(pts_t, ((0, 0), (0, 0), (0, NP - N))).reshape(-1)
    return _render(pts_t, m).reshape(B, 3, IMG, IMG)
